# Initial kernel scaffold; baseline (speedup 1.0000x reference)
import jax
import jax.numpy as jnp
from jax.experimental import pallas as pl


def kernel(features, score, distances, nidxs, row_splits, tidxs):
    raise NotImplementedError("write your pallas kernel here")


# speedup vs baseline: 1.3682x; 1.3682x over previous
import functools

import jax
import jax.numpy as jnp
from jax import lax
from jax.experimental import pallas as pl
from jax.experimental.pallas import tpu as pltpu
from jax.experimental.pallas import tpu_sc as plsc


def _rank_body(seg_len, chunk, s_ref, bg_ref, hier_ref):
    seg = pl.program_id(0)
    offset = seg * seg_len
    s_row = s_ref[0]
    nch = seg_len // chunk
    r_iota = lax.broadcasted_iota(jnp.int32, (chunk, seg_len), 0)
    lane = lax.broadcasted_iota(jnp.int32, (chunk, seg_len), 1)
    s_bcast = jnp.broadcast_to(s_row, (chunk, seg_len))

    acc = jnp.zeros((1, seg_len), jnp.int32)
    for t in range(nch):
        jidx = r_iota + (t * chunk)
        oh = lane == jidx
        scol = jnp.sum(jnp.where(oh, s_bcast, 0.0), axis=1, keepdims=True)
        before = (scol > s_row) | ((scol == s_row) & (jidx < lane))
        acc = acc + jnp.sum(before.astype(jnp.int32), axis=0, keepdims=True)
    ranks = acc
    bg_ref[0] = ranks + offset

    acc2 = jnp.zeros((1, seg_len), jnp.int32)
    for t in range(nch):
        iidx = r_iota + (t * chunk)
        oh = lane == iidx
        rcol = jnp.sum(jnp.where(oh, ranks, 0), axis=1, keepdims=True)
        eq = rcol == lane
        acc2 = acc2 + jnp.sum(jnp.where(eq, iidx, 0), axis=0, keepdims=True)
    hier_ref[0] = acc2 + offset


def _tc_sort(s2d):
    num_seg, seg_len = s2d.shape
    body = functools.partial(_rank_body, seg_len, 256)
    spec = pl.BlockSpec((1, 1, seg_len), lambda k: (k, 0, 0))
    bg3, hier3 = pl.pallas_call(
        body,
        grid=(num_seg,),
        in_specs=[spec],
        out_specs=[spec, spec],
        out_shape=[
            jax.ShapeDtypeStruct((num_seg, 1, seg_len), jnp.int32),
            jax.ShapeDtypeStruct((num_seg, 1, seg_len), jnp.int32),
        ],
    )(s2d.reshape(num_seg, 1, seg_len))
    return bg3.reshape(num_seg, seg_len), hier3.reshape(num_seg, seg_len)


def _make_sc_gather(n, f, lanes, n_workers, chunk):
    rows_per_w = n // n_workers
    nch = rows_per_w // chunk
    mesh = plsc.VectorSubcoreMesh(core_axis_name="c", subcore_axis_name="s")
    nc = mesh.num_cores

    @functools.partial(
        pl.kernel,
        out_type=jax.ShapeDtypeStruct((n, 2 * f), jnp.float32),
        mesh=mesh,
        scratch_types=[
            pltpu.VMEM((nch, chunk), jnp.int32),
            pltpu.VMEM((nch, chunk), jnp.int32),
            pltpu.VMEM((chunk, f), jnp.float32),
            pltpu.VMEM((chunk, f), jnp.float32),
            pltpu.SemaphoreType.DMA,
            pltpu.SemaphoreType.DMA,
        ],
    )
    def sc_gather(features_hbm, nbr1_hbm, hier_hbm, out_hbm,
                  hier_v, nbrs_v, fs_buf, fn_buf, sem_s, sem_n):
        wid = lax.axis_index("s") * nc + lax.axis_index("c")
        base = wid * rows_per_w
        pltpu.sync_copy(hier_hbm.at[pl.ds(wid * nch, nch)], hier_v)
        cps = [pltpu.async_copy(nbr1_hbm.at[hier_v.at[t]], nbrs_v.at[t], sem_n)
               for t in range(nch)]
        for cp in cps:
            cp.wait()
        for t in range(nch):
            g1 = pltpu.async_copy(features_hbm.at[hier_v.at[t]], fs_buf, sem_s)
            g2 = pltpu.async_copy(features_hbm.at[nbrs_v.at[t]], fn_buf, sem_n)
            g1.wait()
            g2.wait()
            row0 = base + t * chunk
            pltpu.sync_copy(fs_buf, out_hbm.at[pl.ds(row0, chunk), pl.ds(0, f)])
            pltpu.sync_copy(fn_buf, out_hbm.at[pl.ds(row0, chunk), pl.ds(f, f)])

    return sc_gather


def kernel(features, score, distances, nidxs, row_splits, tidxs):
    n, f = features.shape
    num_seg = row_splits.shape[0] - 1
    seg_len = n // num_seg

    s2d = jax.nn.sigmoid(score)[:, 0].reshape(num_seg, seg_len)
    bg2d, hier2d = _tc_sort(s2d)

    lanes = 16
    n_workers = 32
    chunk = 128
    hier_c = hier2d.reshape(n // chunk, chunk)
    nbr1 = nidxs[:, 1]

    sc = _make_sc_gather(n, f, lanes, n_workers, chunk)
    out_features = sc(features, nbr1, hier_c)

    backgather = bg2d.reshape(n, 1)
    return out_features, row_splits, backgather


# speedup vs baseline: 1.6247x; 1.1875x over previous
import functools

import jax
import jax.numpy as jnp
from jax import lax
from jax.experimental import pallas as pl
from jax.experimental.pallas import tpu as pltpu
from jax.experimental.pallas import tpu_sc as plsc


def _rank_body(seg_len, chunk, s_ref, st_ref, bg_ref):
    seg = pl.program_id(0)
    offset = seg * seg_len
    s_row = s_ref[0]
    s_cols = st_ref[0]
    nch = seg_len // chunk
    r_iota = lax.broadcasted_iota(jnp.int32, (chunk, seg_len), 0)
    lane = lax.broadcasted_iota(jnp.int32, (chunk, seg_len), 1)

    acc = jnp.zeros((1, seg_len), jnp.int32)
    for t in range(nch):
        scol = s_cols[:, t:t + 1]
        jidx = r_iota + (t * chunk)
        before = (scol > s_row) | ((scol == s_row) & (jidx < lane))
        acc = acc + jnp.sum(before.astype(jnp.int32), axis=0, keepdims=True)
    bg_ref[0] = acc + offset


def _tc_rank(s2d, chunk):
    num_seg, seg_len = s2d.shape
    nch = seg_len // chunk
    body = functools.partial(_rank_body, seg_len, chunk)
    st = s2d.reshape(num_seg, nch, chunk).transpose(0, 2, 1)
    bg3 = pl.pallas_call(
        body,
        grid=(num_seg,),
        in_specs=[
            pl.BlockSpec((1, 1, seg_len), lambda k: (k, 0, 0)),
            pl.BlockSpec((1, chunk, nch), lambda k: (k, 0, 0)),
        ],
        out_specs=pl.BlockSpec((1, 1, seg_len), lambda k: (k, 0, 0)),
        out_shape=jax.ShapeDtypeStruct((num_seg, 1, seg_len), jnp.int32),
    )(s2d.reshape(num_seg, 1, seg_len), st)
    return bg3.reshape(num_seg, seg_len)


def _make_sc_scatter(n, f, n_workers, chunk):
    rows_per_w = n // n_workers
    nch = rows_per_w // chunk
    mesh = plsc.VectorSubcoreMesh(core_axis_name="c", subcore_axis_name="s")
    nc = mesh.num_cores

    @functools.partial(
        pl.kernel,
        out_type=jax.ShapeDtypeStruct((n, 2 * f), jnp.float32),
        mesh=mesh,
        scratch_types=[
            pltpu.VMEM((nch, chunk), jnp.int32),
            pltpu.VMEM((nch, chunk), jnp.int32),
            pltpu.VMEM((chunk, f), jnp.float32),
            pltpu.VMEM((chunk, f), jnp.float32),
            pltpu.SemaphoreType.DMA,
            pltpu.SemaphoreType.DMA,
        ],
    )
    def sc_scatter(features_hbm, nbr1_hbm, bg_hbm, out_hbm,
                   bg_v, nbr_v, fs_buf, fn_buf, sem_s, sem_n):
        wid = lax.axis_index("s") * nc + lax.axis_index("c")
        base = wid * rows_per_w
        pltpu.sync_copy(bg_hbm.at[pl.ds(wid * nch, nch)], bg_v)
        pltpu.sync_copy(nbr1_hbm.at[pl.ds(wid * nch, nch)], nbr_v)
        for t in range(nch):
            g1 = pltpu.async_copy(
                features_hbm.at[pl.ds(base + t * chunk, chunk)], fs_buf, sem_s)
            g2 = pltpu.async_copy(
                features_hbm.at[nbr_v.at[t]], fn_buf, sem_n)
            g1.wait()
            g2.wait()
            s1 = pltpu.async_copy(
                fs_buf, out_hbm.at[bg_v.at[t], pl.ds(0, f)], sem_s)
            s2 = pltpu.async_copy(
                fn_buf, out_hbm.at[bg_v.at[t], pl.ds(f, f)], sem_n)
            s1.wait()
            s2.wait()

    return sc_scatter


def kernel(features, score, distances, nidxs, row_splits, tidxs):
    n, f = features.shape
    num_seg = row_splits.shape[0] - 1
    seg_len = n // num_seg

    s2d = jax.nn.sigmoid(score)[:, 0].reshape(num_seg, seg_len)
    bg2d = _tc_rank(s2d, 256)

    n_workers = 32
    chunk = 128
    bg_c = bg2d.reshape(n // chunk, chunk)
    nbr_c = nidxs[:, 1].reshape(n // chunk, chunk)

    sc = _make_sc_scatter(n, f, n_workers, chunk)
    out_features = sc(features, nbr_c, bg_c)

    backgather = bg2d.reshape(n, 1)
    return out_features, row_splits, backgather
